# baseline (speedup 1.0000x reference)
import functools

import jax
import jax.numpy as jnp
from jax import lax
from jax.experimental import pallas as pl
from jax.experimental.pallas import tpu as pltpu
from jax.experimental.pallas import tpu_sc as plsc

N_NODES = 10000
N_EDGES = 160000
D_IN = 256
D_HID = 256
HALF = 128

EB = 128
NBLK = N_EDGES // EB
SUP = 8
NSUP = NBLK // SUP
NLEFT = NBLK - NSUP * SUP
NS = 16
NC = 2
SPAN = 624
LAST_SPAN = N_NODES - (NS - 1) * SPAN
SUP_STEPS = (NSUP + NS - 1) // NS


def _per_tile_span(sid, fn):
    @pl.when(sid < NS - 1)
    def _():
        fn(sid * SPAN, SPAN)

    @pl.when(sid == NS - 1)
    def _():
        fn((NS - 1) * SPAN, LAST_SPAN)


@functools.cache
def _sc_kernels():
    mesh = plsc.VectorSubcoreMesh(core_axis_name="c", subcore_axis_name="s")

    half_sup = NSUP // NC
    deg_steps = (half_sup + NS - 1) // NS

    @functools.partial(
        pl.kernel,
        mesh=mesh,
        out_type=[
            jax.ShapeDtypeStruct((N_NODES, HALF), jnp.float32),
            jax.ShapeDtypeStruct((N_NODES, HALF), jnp.float32),
        ],
        scratch_types=[
            pltpu.VMEM((NC, SUP, EB), jnp.int32),
            pltpu.VMEM((EB, HALF), jnp.float32),
            pltpu.VMEM_SHARED((N_NODES, HALF), jnp.float32),
            pltpu.SemaphoreType.DMA,
        ],
    )
    def deg_kernel(dst2, ones_hbm, zeros_hbm, d0_out, d1_out, dst8x2, onesv,
                   deg_sh, sem_s):
        cid = lax.axis_index("c")
        sid = lax.axis_index("s")
        pltpu.sync_copy(ones_hbm, onesv)
        _per_tile_span(sid, lambda base, n: pltpu.sync_copy(
            zeros_hbm.at[pl.ds(0, n)], deg_sh.at[pl.ds(base, n)]))
        plsc.subcore_barrier()

        def wait_one():
            pltpu.make_async_copy(onesv, deg_sh.at[dst8x2.at[0].at[0]],
                                  sem_s).wait()

        def step(r, carry):
            s_local = sid + r * NS

            @pl.when(s_local < half_sup)
            def _():
                @pl.when(r >= 2)
                def _():
                    for _ in range(SUP):
                        wait_one()

                dst8 = dst8x2.at[r % 2]
                blk0 = (cid * half_sup + s_local) * SUP
                pltpu.sync_copy(dst2.at[pl.ds(blk0, SUP)], dst8)
                for j in range(SUP):
                    pltpu.async_copy(onesv, deg_sh.at[dst8.at[j]], sem_s,
                                     add=True)

            return carry

        lax.fori_loop(0, deg_steps, step, 0)
        for _ in range(2 * SUP):
            wait_one()

        @pl.when((cid == 1) & (sid == NS - 1))
        def _():
            dst8 = dst8x2.at[0]
            pltpu.sync_copy(dst2.at[pl.ds(NSUP * SUP, NLEFT)],
                            dst8.at[pl.ds(0, NLEFT)])
            for j in range(NLEFT):
                pltpu.async_copy(onesv, deg_sh.at[dst8.at[j]], sem_s,
                                 add=True)
            for _ in range(NLEFT):
                wait_one()

        plsc.subcore_barrier()

        @pl.when(cid == 0)
        def _():
            _per_tile_span(sid, lambda base, n: pltpu.sync_copy(
                deg_sh.at[pl.ds(base, n)], d0_out.at[pl.ds(base, n)]))

        @pl.when(cid == 1)
        def _():
            _per_tile_span(sid, lambda base, n: pltpu.sync_copy(
                deg_sh.at[pl.ds(base, n)], d1_out.at[pl.ds(base, n)]))

    @functools.partial(
        pl.kernel,
        mesh=mesh,
        out_type=[
            jax.ShapeDtypeStruct((N_NODES, HALF), jnp.float32),
            jax.ShapeDtypeStruct((N_NODES, HALF), jnp.float32),
        ],
        scratch_types=[
            pltpu.VMEM((NC, SUP, EB), jnp.int32),
            pltpu.VMEM((NC, SUP, EB), jnp.int32),
            pltpu.VMEM((2, EB, HALF), jnp.float32),
            pltpu.VMEM_SHARED((N_NODES, HALF), jnp.float32),
            pltpu.SemaphoreType.DMA,
            pltpu.SemaphoreType.DMA,
            pltpu.SemaphoreType.DMA,
        ],
    )
    def agg_kernel(ylo, yhi, src2, dst2, zlo, zhi, src8x2, dst8x2, rows, zsh,
                   sem_g, sem_s, sem_i):
        cid = lax.axis_index("c")
        sid = lax.axis_index("s")

        def wait_one_scatter():
            pltpu.make_async_copy(rows.at[0], zsh.at[dst8x2.at[0].at[0]],
                                  sem_s).wait()

        def wait_one_gather(ytab):
            pltpu.make_async_copy(ytab.at[src8x2.at[0].at[0]], rows.at[0],
                                  sem_g).wait()

        def run_edges(ytab):
            def wait_one_idx():
                pltpu.make_async_copy(src2.at[pl.ds(0, SUP)],
                                      src8x2.at[0], sem_i).wait()

            def super_step(r, carry):
                s = sid + r * NS

                @pl.when(s < NSUP)
                def _():
                    src8 = src8x2.at[r % 2]
                    dst8 = dst8x2.at[r % 2]
                    dst8p = dst8x2.at[(r + 1) % 2]

                    @pl.when(r >= 1)
                    def _():
                        wait_one_idx()
                        wait_one_idx()

                    for j in range(SUP):
                        g = r * SUP + j
                        sb = g % 2

                        @pl.when(g >= 2)
                        def _():
                            wait_one_scatter()

                        pltpu.async_copy(ytab.at[src8.at[j]], rows.at[sb],
                                         sem_g)
                        if j >= 1:
                            wait_one_gather(ytab)
                            pltpu.async_copy(rows.at[1 - sb],
                                             zsh.at[dst8.at[j - 1]],
                                             sem_s, add=True)
                        else:
                            @pl.when(g >= 1)
                            def _():
                                wait_one_gather(ytab)
                                pltpu.async_copy(rows.at[1 - sb],
                                                 zsh.at[dst8p.at[SUP - 1]],
                                                 sem_s, add=True)
                        if j == 2:
                            sn = s + NS

                            @pl.when(sn < NSUP)
                            def _():
                                blkn = sn * SUP
                                pltpu.async_copy(src2.at[pl.ds(blkn, SUP)],
                                                 src8x2.at[(r + 1) % 2],
                                                 sem_i)
                                pltpu.async_copy(dst2.at[pl.ds(blkn, SUP)],
                                                 dst8x2.at[(r + 1) % 2],
                                                 sem_i)

                return carry

            @pl.when(sid < NSUP)
            def _():
                pltpu.sync_copy(src2.at[pl.ds(sid * SUP, SUP)], src8x2.at[0])
                pltpu.sync_copy(dst2.at[pl.ds(sid * SUP, SUP)], dst8x2.at[0])

            lax.fori_loop(0, SUP_STEPS, super_step, 0)
            wait_one_gather(ytab)
            last_r = lax.rem(
                jnp.where(sid < NSUP - (SUP_STEPS - 1) * NS, SUP_STEPS - 1,
                          SUP_STEPS - 2), 2)
            pltpu.async_copy(rows.at[1], zsh.at[dst8x2.at[last_r].at[SUP - 1]],
                             sem_s, add=True)
            wait_one_scatter()
            wait_one_scatter()

            @pl.when(sid == NS - 1)
            def _():
                src8 = src8x2.at[0]
                dst8 = dst8x2.at[0]
                pltpu.sync_copy(src2.at[pl.ds(NSUP * SUP, NLEFT)],
                                src8.at[pl.ds(0, NLEFT)])
                pltpu.sync_copy(dst2.at[pl.ds(NSUP * SUP, NLEFT)],
                                dst8.at[pl.ds(0, NLEFT)])
                d0 = pltpu.async_copy(ytab.at[src8.at[0]], rows.at[0], sem_g)
                d1 = pltpu.async_copy(ytab.at[src8.at[1]], rows.at[1], sem_g)
                d0.wait()
                d1.wait()
                pltpu.async_copy(rows.at[0], zsh.at[dst8.at[0]], sem_s,
                                 add=True)
                pltpu.async_copy(rows.at[1], zsh.at[dst8.at[1]], sem_s,
                                 add=True)
                wait_one_scatter()
                wait_one_scatter()

        @pl.when(cid == 0)
        def _():
            _per_tile_span(sid, lambda base, n: pltpu.sync_copy(
                ylo.at[pl.ds(base, n)], zsh.at[pl.ds(base, n)]))
            plsc.subcore_barrier()
            run_edges(ylo)
            plsc.subcore_barrier()

        @pl.when(cid == 1)
        def _():
            _per_tile_span(sid, lambda base, n: pltpu.sync_copy(
                yhi.at[pl.ds(base, n)], zsh.at[pl.ds(base, n)]))
            plsc.subcore_barrier()
            run_edges(yhi)
            plsc.subcore_barrier()

        @pl.when(cid == 0)
        def _():
            _per_tile_span(sid, lambda base, n: pltpu.sync_copy(
                zsh.at[pl.ds(base, n)], zlo.at[pl.ds(base, n)]))

        @pl.when(cid == 1)
        def _():
            _per_tile_span(sid, lambda base, n: pltpu.sync_copy(
                zsh.at[pl.ds(base, n)], zhi.at[pl.ds(base, n)]))

    return deg_kernel, agg_kernel


_R = 10000


def _tc1_body(d0_ref, d1_ref, x_ref, w_ref, ylo_ref, yhi_ref, dis_ref):
    dis = lax.rsqrt(d0_ref[:, 0:1] + d1_ref[:, 0:1] + 1.0)
    xw = jnp.dot(x_ref[...], w_ref[...], preferred_element_type=jnp.float32)
    y = xw * dis
    ylo_ref[...] = y[:, :HALF]
    yhi_ref[...] = y[:, HALF:]
    dis_ref[...] = dis


def _tc1(d0, d1, x, W0):
    return pl.pallas_call(
        _tc1_body,
        grid=(N_NODES // _R,),
        in_specs=[
            pl.BlockSpec((_R, HALF), lambda i: (i, 0)),
            pl.BlockSpec((_R, HALF), lambda i: (i, 0)),
            pl.BlockSpec((_R, D_IN), lambda i: (i, 0)),
            pl.BlockSpec((D_IN, D_HID), lambda i: (0, 0)),
        ],
        out_specs=[
            pl.BlockSpec((_R, HALF), lambda i: (i, 0)),
            pl.BlockSpec((_R, HALF), lambda i: (i, 0)),
            pl.BlockSpec((_R, 1), lambda i: (i, 0)),
        ],
        out_shape=[
            jax.ShapeDtypeStruct((N_NODES, HALF), jnp.float32),
            jax.ShapeDtypeStruct((N_NODES, HALF), jnp.float32),
            jax.ShapeDtypeStruct((N_NODES, 1), jnp.float32),
        ],
    )(d0, d1, x, W0)


def _tc2_body(dis_ref, zlo_ref, zhi_ref, b_ref, w_ref, ylo_ref, yhi_ref):
    dis = dis_ref[...]
    h_lo = jnp.maximum(zlo_ref[...] * dis + b_ref[:, :HALF], 0.0)
    h_hi = jnp.maximum(zhi_ref[...] * dis + b_ref[:, HALF:], 0.0)
    res = jnp.dot(h_lo, w_ref[:HALF, :], preferred_element_type=jnp.float32)
    res = res + jnp.dot(h_hi, w_ref[HALF:, :], preferred_element_type=jnp.float32)
    y = res * dis
    ylo_ref[...] = y[:, :HALF]
    yhi_ref[...] = y[:, HALF:]


def _tc2(dis, zlo, zhi, b0, W1):
    return pl.pallas_call(
        _tc2_body,
        grid=(N_NODES // _R,),
        in_specs=[
            pl.BlockSpec((_R, 1), lambda i: (i, 0)),
            pl.BlockSpec((_R, HALF), lambda i: (i, 0)),
            pl.BlockSpec((_R, HALF), lambda i: (i, 0)),
            pl.BlockSpec((1, D_HID), lambda i: (0, 0)),
            pl.BlockSpec((D_HID, D_HID), lambda i: (0, 0)),
        ],
        out_specs=[
            pl.BlockSpec((_R, HALF), lambda i: (i, 0)),
            pl.BlockSpec((_R, HALF), lambda i: (i, 0)),
        ],
        out_shape=[
            jax.ShapeDtypeStruct((N_NODES, HALF), jnp.float32),
            jax.ShapeDtypeStruct((N_NODES, HALF), jnp.float32),
        ],
    )(dis, zlo, zhi, b0, W1)


def _tc3_body(dis_ref, zlo_ref, zhi_ref, b_ref, w_ref, bout_ref, out_ref):
    dis = dis_ref[...]
    h_lo = jnp.maximum(zlo_ref[...] * dis + b_ref[:, :HALF], 0.0)
    h_hi = jnp.maximum(zhi_ref[...] * dis + b_ref[:, HALF:], 0.0)
    res = jnp.dot(h_lo, w_ref[:HALF, :], preferred_element_type=jnp.float32)
    res = res + jnp.dot(h_hi, w_ref[HALF:, :], preferred_element_type=jnp.float32)
    out_ref[...] = res + bout_ref[...]


def _tc3(dis, zlo, zhi, b1, Wout, bout):
    return pl.pallas_call(
        _tc3_body,
        grid=(N_NODES // _R,),
        in_specs=[
            pl.BlockSpec((_R, 1), lambda i: (i, 0)),
            pl.BlockSpec((_R, HALF), lambda i: (i, 0)),
            pl.BlockSpec((_R, HALF), lambda i: (i, 0)),
            pl.BlockSpec((1, D_HID), lambda i: (0, 0)),
            pl.BlockSpec((D_HID, 1), lambda i: (0, 0)),
            pl.BlockSpec((1, 1), lambda i: (0, 0)),
        ],
        out_specs=pl.BlockSpec((_R, 1), lambda i: (i, 0)),
        out_shape=jax.ShapeDtypeStruct((N_NODES, 1), jnp.float32),
    )(dis, zlo, zhi, b1, Wout, bout)


def kernel(x, edge_index, W0, b0, W1, b1, Wout, bout):
    deg_kernel, agg_kernel = _sc_kernels()
    src2 = edge_index[0].astype(jnp.int32).reshape(NBLK, EB)
    dst2 = edge_index[1].astype(jnp.int32).reshape(NBLK, EB)
    ones = jnp.ones((EB, HALF), jnp.float32)
    zeros = jnp.zeros((LAST_SPAN, HALF), jnp.float32)

    d0, d1 = deg_kernel(dst2, ones, zeros)
    ylo, yhi, dis = _tc1(d0, d1, x, W0)
    zlo, zhi = agg_kernel(ylo, yhi, src2, dst2)
    y2lo, y2hi = _tc2(dis, zlo, zhi, b0.reshape(1, D_HID), W1)
    z2lo, z2hi = agg_kernel(y2lo, y2hi, src2, dst2)
    return _tc3(dis, z2lo, z2hi, b1.reshape(1, D_HID), Wout, bout.reshape(1, 1))


# speedup vs baseline: 1.0193x; 1.0193x over previous
import functools

import jax
import jax.numpy as jnp
from jax import lax
from jax.experimental import pallas as pl
from jax.experimental.pallas import tpu as pltpu
from jax.experimental.pallas import tpu_sc as plsc

N_NODES = 10000
N_EDGES = 160000
D_IN = 256
D_HID = 256
HALF = 128

EB = 128
NBLK = N_EDGES // EB
SUP = 8
NSUP = NBLK // SUP
NLEFT = NBLK - NSUP * SUP
NS = 16
NC = 2
SPAN = 624
LAST_SPAN = N_NODES - (NS - 1) * SPAN
SUP_STEPS = (NSUP + NS - 1) // NS


def _per_tile_span(sid, fn):
    @pl.when(sid < NS - 1)
    def _():
        fn(sid * SPAN, SPAN)

    @pl.when(sid == NS - 1)
    def _():
        fn((NS - 1) * SPAN, LAST_SPAN)


@functools.cache
def _sc_kernels():
    mesh = plsc.VectorSubcoreMesh(core_axis_name="c", subcore_axis_name="s")

    half_sup = NSUP // NC
    deg_steps = (half_sup + NS - 1) // NS

    @functools.partial(
        pl.kernel,
        mesh=mesh,
        out_type=[
            jax.ShapeDtypeStruct((N_NODES, HALF), jnp.float32),
            jax.ShapeDtypeStruct((N_NODES, HALF), jnp.float32),
        ],
        scratch_types=[
            pltpu.VMEM((NC, SUP, EB), jnp.int32),
            pltpu.VMEM((EB, HALF), jnp.float32),
            pltpu.VMEM_SHARED((N_NODES, HALF), jnp.float32),
            pltpu.SemaphoreType.DMA,
        ],
    )
    def deg_kernel(dst2, ones_hbm, zeros_hbm, d0_out, d1_out, dst8x2, onesv,
                   deg_sh, sem_s):
        cid = lax.axis_index("c")
        sid = lax.axis_index("s")
        pltpu.sync_copy(ones_hbm, onesv)
        _per_tile_span(sid, lambda base, n: pltpu.sync_copy(
            zeros_hbm.at[pl.ds(0, n)], deg_sh.at[pl.ds(base, n)]))
        plsc.subcore_barrier()

        def wait_one():
            pltpu.make_async_copy(onesv, deg_sh.at[dst8x2.at[0].at[0]],
                                  sem_s).wait()

        def step(r, carry):
            s_local = sid + r * NS

            @pl.when(s_local < half_sup)
            def _():
                @pl.when(r >= 2)
                def _():
                    for _ in range(SUP):
                        wait_one()

                dst8 = dst8x2.at[r % 2]
                blk0 = (cid * half_sup + s_local) * SUP
                pltpu.sync_copy(dst2.at[pl.ds(blk0, SUP)], dst8)
                for j in range(SUP):
                    pltpu.async_copy(onesv, deg_sh.at[dst8.at[j]], sem_s,
                                     add=True)

            return carry

        lax.fori_loop(0, deg_steps, step, 0)
        for _ in range(2 * SUP):
            wait_one()

        @pl.when((cid == 1) & (sid == NS - 1))
        def _():
            dst8 = dst8x2.at[0]
            pltpu.sync_copy(dst2.at[pl.ds(NSUP * SUP, NLEFT)],
                            dst8.at[pl.ds(0, NLEFT)])
            for j in range(NLEFT):
                pltpu.async_copy(onesv, deg_sh.at[dst8.at[j]], sem_s,
                                 add=True)
            for _ in range(NLEFT):
                wait_one()

        plsc.subcore_barrier()

        @pl.when(cid == 0)
        def _():
            _per_tile_span(sid, lambda base, n: pltpu.sync_copy(
                deg_sh.at[pl.ds(base, n)], d0_out.at[pl.ds(base, n)]))

        @pl.when(cid == 1)
        def _():
            _per_tile_span(sid, lambda base, n: pltpu.sync_copy(
                deg_sh.at[pl.ds(base, n)], d1_out.at[pl.ds(base, n)]))

    @functools.partial(
        pl.kernel,
        mesh=mesh,
        out_type=[
            jax.ShapeDtypeStruct((N_NODES, HALF), jnp.float32),
            jax.ShapeDtypeStruct((N_NODES, HALF), jnp.float32),
        ],
        scratch_types=[
            pltpu.VMEM((NC, SUP, EB), jnp.int32),
            pltpu.VMEM((NC, SUP, EB), jnp.int32),
            pltpu.VMEM((2, EB, HALF), jnp.float32),
            pltpu.VMEM_SHARED((N_NODES, HALF), jnp.float32),
            pltpu.SemaphoreType.DMA,
            pltpu.SemaphoreType.DMA,
            pltpu.SemaphoreType.DMA,
        ],
    )
    def agg_kernel(ylo, yhi, src2, dst2, zlo, zhi, src8x2, dst8x2, rows, zsh,
                   sem_g, sem_s, sem_i):
        cid = lax.axis_index("c")
        sid = lax.axis_index("s")

        def wait_one_scatter():
            pltpu.make_async_copy(rows.at[0], zsh.at[dst8x2.at[0].at[0]],
                                  sem_s).wait()

        def wait_one_gather(ytab):
            pltpu.make_async_copy(ytab.at[src8x2.at[0].at[0]], rows.at[0],
                                  sem_g).wait()

        def run_edges(ytab):
            def wait_one_idx():
                pltpu.make_async_copy(src2.at[pl.ds(0, SUP)],
                                      src8x2.at[0], sem_i).wait()

            def super_step(r, carry):
                s = sid + r * NS

                @pl.when(s < NSUP)
                def _():
                    src8 = src8x2.at[r % 2]
                    dst8 = dst8x2.at[r % 2]
                    dst8p = dst8x2.at[(r + 1) % 2]

                    @pl.when(r >= 1)
                    def _():
                        wait_one_idx()
                        wait_one_idx()

                    for j in range(SUP):
                        g = r * SUP + j
                        sb = g % 2

                        @pl.when(g >= 2)
                        def _():
                            wait_one_scatter()

                        pltpu.async_copy(ytab.at[src8.at[j]], rows.at[sb],
                                         sem_g)
                        if j >= 1:
                            wait_one_gather(ytab)
                            pltpu.async_copy(rows.at[1 - sb],
                                             zsh.at[dst8.at[j - 1]],
                                             sem_s, add=True)
                        else:
                            @pl.when(g >= 1)
                            def _():
                                wait_one_gather(ytab)
                                pltpu.async_copy(rows.at[1 - sb],
                                                 zsh.at[dst8p.at[SUP - 1]],
                                                 sem_s, add=True)
                        if j == 2:
                            sn = s + NS

                            @pl.when(sn < NSUP)
                            def _():
                                blkn = sn * SUP
                                pltpu.async_copy(src2.at[pl.ds(blkn, SUP)],
                                                 src8x2.at[(r + 1) % 2],
                                                 sem_i)
                                pltpu.async_copy(dst2.at[pl.ds(blkn, SUP)],
                                                 dst8x2.at[(r + 1) % 2],
                                                 sem_i)

                return carry

            @pl.when(sid < NSUP)
            def _():
                pltpu.sync_copy(src2.at[pl.ds(sid * SUP, SUP)], src8x2.at[0])
                pltpu.sync_copy(dst2.at[pl.ds(sid * SUP, SUP)], dst8x2.at[0])

            lax.fori_loop(0, SUP_STEPS, super_step, 0)
            wait_one_gather(ytab)
            last_r = lax.rem(
                jnp.where(sid < NSUP - (SUP_STEPS - 1) * NS, SUP_STEPS - 1,
                          SUP_STEPS - 2), 2)
            pltpu.async_copy(rows.at[1], zsh.at[dst8x2.at[last_r].at[SUP - 1]],
                             sem_s, add=True)
            wait_one_scatter()
            wait_one_scatter()

            @pl.when(sid == NS - 1)
            def _():
                src8 = src8x2.at[0]
                dst8 = dst8x2.at[0]
                pltpu.sync_copy(src2.at[pl.ds(NSUP * SUP, NLEFT)],
                                src8.at[pl.ds(0, NLEFT)])
                pltpu.sync_copy(dst2.at[pl.ds(NSUP * SUP, NLEFT)],
                                dst8.at[pl.ds(0, NLEFT)])
                d0 = pltpu.async_copy(ytab.at[src8.at[0]], rows.at[0], sem_g)
                d1 = pltpu.async_copy(ytab.at[src8.at[1]], rows.at[1], sem_g)
                d0.wait()
                d1.wait()
                pltpu.async_copy(rows.at[0], zsh.at[dst8.at[0]], sem_s,
                                 add=True)
                pltpu.async_copy(rows.at[1], zsh.at[dst8.at[1]], sem_s,
                                 add=True)
                wait_one_scatter()
                wait_one_scatter()

        @pl.when(cid == 0)
        def _():
            _per_tile_span(sid, lambda base, n: pltpu.sync_copy(
                ylo.at[pl.ds(base, n)], zsh.at[pl.ds(base, n)]))
            plsc.subcore_barrier()
            run_edges(ylo)
            plsc.subcore_barrier()

        @pl.when(cid == 1)
        def _():
            _per_tile_span(sid, lambda base, n: pltpu.sync_copy(
                yhi.at[pl.ds(base, n)], zsh.at[pl.ds(base, n)]))
            plsc.subcore_barrier()
            run_edges(yhi)
            plsc.subcore_barrier()

        @pl.when(cid == 0)
        def _():
            _per_tile_span(sid, lambda base, n: pltpu.sync_copy(
                zsh.at[pl.ds(base, n)], zlo.at[pl.ds(base, n)]))

        @pl.when(cid == 1)
        def _():
            _per_tile_span(sid, lambda base, n: pltpu.sync_copy(
                zsh.at[pl.ds(base, n)], zhi.at[pl.ds(base, n)]))

    return deg_kernel, agg_kernel


_R = 5000


def _tc1_body(d0_ref, d1_ref, x_ref, w_ref, ylo_ref, yhi_ref, dis_ref):
    dis = lax.rsqrt(d0_ref[:, 0:1] + d1_ref[:, 0:1] + 1.0)
    xw = jnp.dot(x_ref[...], w_ref[...], preferred_element_type=jnp.float32)
    y = xw * dis
    ylo_ref[...] = y[:, :HALF]
    yhi_ref[...] = y[:, HALF:]
    dis_ref[...] = dis


def _tc1(d0, d1, x, W0):
    return pl.pallas_call(
        _tc1_body,
        grid=(N_NODES // _R,),
        in_specs=[
            pl.BlockSpec((_R, HALF), lambda i: (i, 0)),
            pl.BlockSpec((_R, HALF), lambda i: (i, 0)),
            pl.BlockSpec((_R, D_IN), lambda i: (i, 0)),
            pl.BlockSpec((D_IN, D_HID), lambda i: (0, 0)),
        ],
        out_specs=[
            pl.BlockSpec((_R, HALF), lambda i: (i, 0)),
            pl.BlockSpec((_R, HALF), lambda i: (i, 0)),
            pl.BlockSpec((_R, 1), lambda i: (i, 0)),
        ],
        out_shape=[
            jax.ShapeDtypeStruct((N_NODES, HALF), jnp.float32),
            jax.ShapeDtypeStruct((N_NODES, HALF), jnp.float32),
            jax.ShapeDtypeStruct((N_NODES, 1), jnp.float32),
        ],
    )(d0, d1, x, W0)


def _tc2_body(dis_ref, zlo_ref, zhi_ref, b_ref, w_ref, ylo_ref, yhi_ref):
    dis = dis_ref[...]
    h_lo = jnp.maximum(zlo_ref[...] * dis + b_ref[:, :HALF], 0.0)
    h_hi = jnp.maximum(zhi_ref[...] * dis + b_ref[:, HALF:], 0.0)
    res = jnp.dot(h_lo, w_ref[:HALF, :], preferred_element_type=jnp.float32)
    res = res + jnp.dot(h_hi, w_ref[HALF:, :], preferred_element_type=jnp.float32)
    y = res * dis
    ylo_ref[...] = y[:, :HALF]
    yhi_ref[...] = y[:, HALF:]


def _tc2(dis, zlo, zhi, b0, W1):
    return pl.pallas_call(
        _tc2_body,
        grid=(N_NODES // _R,),
        in_specs=[
            pl.BlockSpec((_R, 1), lambda i: (i, 0)),
            pl.BlockSpec((_R, HALF), lambda i: (i, 0)),
            pl.BlockSpec((_R, HALF), lambda i: (i, 0)),
            pl.BlockSpec((1, D_HID), lambda i: (0, 0)),
            pl.BlockSpec((D_HID, D_HID), lambda i: (0, 0)),
        ],
        out_specs=[
            pl.BlockSpec((_R, HALF), lambda i: (i, 0)),
            pl.BlockSpec((_R, HALF), lambda i: (i, 0)),
        ],
        out_shape=[
            jax.ShapeDtypeStruct((N_NODES, HALF), jnp.float32),
            jax.ShapeDtypeStruct((N_NODES, HALF), jnp.float32),
        ],
    )(dis, zlo, zhi, b0, W1)


def _tc3_body(dis_ref, zlo_ref, zhi_ref, b_ref, w_ref, bout_ref, out_ref):
    dis = dis_ref[...]
    h_lo = jnp.maximum(zlo_ref[...] * dis + b_ref[:, :HALF], 0.0)
    h_hi = jnp.maximum(zhi_ref[...] * dis + b_ref[:, HALF:], 0.0)
    res = jnp.dot(h_lo, w_ref[:HALF, :], preferred_element_type=jnp.float32)
    res = res + jnp.dot(h_hi, w_ref[HALF:, :], preferred_element_type=jnp.float32)
    out_ref[...] = res + bout_ref[...]


def _tc3(dis, zlo, zhi, b1, Wout, bout):
    return pl.pallas_call(
        _tc3_body,
        grid=(N_NODES // _R,),
        in_specs=[
            pl.BlockSpec((_R, 1), lambda i: (i, 0)),
            pl.BlockSpec((_R, HALF), lambda i: (i, 0)),
            pl.BlockSpec((_R, HALF), lambda i: (i, 0)),
            pl.BlockSpec((1, D_HID), lambda i: (0, 0)),
            pl.BlockSpec((D_HID, 1), lambda i: (0, 0)),
            pl.BlockSpec((1, 1), lambda i: (0, 0)),
        ],
        out_specs=pl.BlockSpec((_R, 1), lambda i: (i, 0)),
        out_shape=jax.ShapeDtypeStruct((N_NODES, 1), jnp.float32),
    )(dis, zlo, zhi, b1, Wout, bout)


def kernel(x, edge_index, W0, b0, W1, b1, Wout, bout):
    deg_kernel, agg_kernel = _sc_kernels()
    src2 = edge_index[0].astype(jnp.int32).reshape(NBLK, EB)
    dst2 = edge_index[1].astype(jnp.int32).reshape(NBLK, EB)
    ones = jnp.ones((EB, HALF), jnp.float32)
    zeros = jnp.zeros((LAST_SPAN, HALF), jnp.float32)

    d0, d1 = deg_kernel(dst2, ones, zeros)
    ylo, yhi, dis = _tc1(d0, d1, x, W0)
    zlo, zhi = agg_kernel(ylo, yhi, src2, dst2)
    y2lo, y2hi = _tc2(dis, zlo, zhi, b0.reshape(1, D_HID), W1)
    z2lo, z2hi = agg_kernel(y2lo, y2hi, src2, dst2)
    return _tc3(dis, z2lo, z2hi, b1.reshape(1, D_HID), Wout, bout.reshape(1, 1))
